# baseline (speedup 1.0000x reference)
import jax
import jax.numpy as jnp
from jax import lax
from jax.experimental import pallas as pl
from jax.experimental.pallas import tpu as pltpu
from jax.experimental.pallas import tpu_sc as plsc

_NC = 2
_NS = 16
_L = 16
_CHUNK = 128
_MH = 64
_DW = _MH + _L


def _encode_body(x_ref, we_ref, be_ref, ws_ref, bs_ref, wm_ref, bm_ref,
                 h_ref, sig_ref, msg0_ref, msg1_ref):
    x = x_ref[...]
    h = jnp.dot(x, we_ref[...], preferred_element_type=jnp.float32) + be_ref[...]
    s = jnp.dot(h, ws_ref[...], preferred_element_type=jnp.float32) + bs_ref[...]
    nrm = jnp.sqrt(jnp.sum(s * s, axis=-1, keepdims=True))
    sig_ref[...] = s / jnp.maximum(nrm, 1e-12)
    msg = jnp.maximum(
        jnp.dot(h, wm_ref[...], preferred_element_type=jnp.float32) + bm_ref[...],
        0.0)
    msg0_ref[...] = msg[:, :_MH]
    msg1_ref[...] = msg[:, _MH:]
    h_ref[...] = h


def _encode(x, W_enc, b_enc, W_sig, b_sig, W_msg, b_msg, block_rows):
    n, d_in = x.shape
    d_h = W_enc.shape[1]
    d_sig = W_sig.shape[1]
    grid = (n // block_rows,)
    full = lambda shape: pl.BlockSpec(shape, lambda i: (0, 0))
    return pl.pallas_call(
        _encode_body,
        grid=grid,
        in_specs=[
            pl.BlockSpec((block_rows, d_in), lambda i: (i, 0)),
            full((d_in, d_h)), full((1, d_h)),
            full((d_h, d_sig)), full((1, d_sig)),
            full((d_h, d_h)), full((1, d_h)),
        ],
        out_specs=[
            pl.BlockSpec((block_rows, d_h), lambda i: (i, 0)),
            pl.BlockSpec((block_rows, d_sig), lambda i: (i, 0)),
            pl.BlockSpec((block_rows, _MH), lambda i: (i, 0)),
            pl.BlockSpec((block_rows, _MH), lambda i: (i, 0)),
        ],
        out_shape=[
            jax.ShapeDtypeStruct((n, d_h), jnp.float32),
            jax.ShapeDtypeStruct((n, d_sig), jnp.float32),
            jax.ShapeDtypeStruct((n, _MH), jnp.float32),
            jax.ShapeDtypeStruct((n, _MH), jnp.float32),
        ],
    )(x, W_enc, b_enc.reshape(1, -1), W_sig, b_sig.reshape(1, -1),
      W_msg, b_msg.reshape(1, -1))


def _sc_edge_kernel(n_nodes, n_edges, cpw, d_sig):
    zrows = 200
    nblocks = n_nodes // zrows
    mesh = plsc.VectorSubcoreMesh(core_axis_name="c", subcore_axis_name="s")

    def body(sig_hbm, msg0_hbm, msg1_hbm, src_hbm, dst_hbm, p_hbm, den_hbm,
             src_v, dst_v, ssrc0, sdst0, m0, m1, m2, ex_v, den_v,
             zero_v,
             p_sh,
             ga0, gb0, gm0, gm1, gm2, sp0, sp1, sp2):
        cid = lax.axis_index("c")
        sid = lax.axis_index("s")
        lane = lax.broadcasted_iota(jnp.int32, (_L,), 0)
        msgb = (m0, m1, m2)
        gm = (gm0, gm1, gm2)
        sp = (sp0, sp1, sp2)

        def zfill(r, _):
            z = jnp.zeros((_L,), jnp.float32)
            for q in range(_MH // _L):
                zero_v[r, pl.ds(q * _L, _L)] = z
            return 0
        lax.fori_loop(0, zrows, zfill, 0, unroll=False)

        def dfill(r, _):
            den_v[pl.ds(r * _L, _L)] = jnp.zeros((_L,), jnp.float32)
            return 0
        lax.fori_loop(0, n_nodes // _L, dfill, 0, unroll=False)

        for k in range((nblocks + _NS - 1) // _NS):
            b = sid + _NS * k
            @pl.when(b < nblocks)
            def _():
                pltpu.sync_copy(zero_v, p_sh.at[pl.ds(b * zrows, zrows)])
        plsc.subcore_barrier()

        pltpu.sync_copy(src_hbm.at[sid], src_v)
        pltpu.sync_copy(dst_hbm.at[sid], dst_v)

        ebase = sid * (cpw * _CHUNK)

        def issue_sig_gathers(j):
            pltpu.async_copy(sig_hbm.at[src_v.at[j]], ssrc0, ga0)
            pltpu.async_copy(sig_hbm.at[dst_v.at[j]], sdst0, gb0)

        def issue_msg_gather(j, m):
            @pl.when(cid == 0)
            def _():
                pltpu.async_copy(msg0_hbm.at[dst_v.at[j]], msgb[m], gm[m])
            @pl.when(cid == 1)
            def _():
                pltpu.async_copy(msg1_hbm.at[dst_v.at[j]], msgb[m], gm[m])

        issue_sig_gathers(0)
        issue_msg_gather(0, 0)
        issue_msg_gather(1, 1)

        def do_chunk(j, m):
            pltpu.make_async_copy(sig_hbm.at[src_v.at[j]], ssrc0,
                                  ga0).wait()
            pltpu.make_async_copy(sig_hbm.at[dst_v.at[j]], sdst0,
                                  gb0).wait()

            for g in range(_CHUNK // _L):
                rows = g * _L + lane
                acc = jnp.zeros((_L,), jnp.float32)
                for f in range(d_sig):
                    col = jnp.full((_L,), f, jnp.int32)
                    a = plsc.load_gather(ssrc0, [rows, col])
                    bb = plsc.load_gather(sdst0, [rows, col])
                    acc = acc + a * bb
                eid = ebase + j * _CHUNK + g * _L + lane
                ex = jnp.where(eid < n_edges, jnp.exp(acc), 0.0)
                ex_v[pl.ds(g * _L, _L)] = ex
                sidx = src_v[j, pl.ds(g * _L, _L)]
                plsc.addupdate_scatter(den_v, [sidx], ex)

            @pl.when(j + 1 < cpw)
            def _():
                issue_sig_gathers(j + 1)

            pm = (m + 2) % 3
            @pl.when(j >= 1)
            def _():
                pltpu.make_async_copy(msgb[pm], p_sh.at[src_v.at[j]],
                                      sp[pm]).wait()
            @pl.when(j + 2 < cpw)
            def _():
                issue_msg_gather(j + 2, pm)

            pltpu.make_async_copy(msg0_hbm.at[dst_v.at[j]], msgb[m],
                                  gm[m]).wait()

            def scale_row(r, _):
                spl = plsc.load_gather(ex_v, [jnp.full((_L,), r, jnp.int32)])
                for q in range(_MH // _L):
                    msgb[m][r, pl.ds(q * _L, _L)] = (
                        msgb[m][r, pl.ds(q * _L, _L)] * spl)
                return 0
            lax.fori_loop(0, _CHUNK, scale_row, 0, unroll=4)

            pltpu.async_copy(msgb[m], p_sh.at[src_v.at[j]], sp[m], add=True)

        def block_body(blk, _):
            for t in range(6):
                do_chunk(6 * blk + t, t % 3)
            return 0

        lax.fori_loop(0, cpw // 6, block_body, 0, unroll=False)
        pltpu.make_async_copy(msgb[(cpw - 1) % 3],
                              p_sh.at[src_v.at[cpw - 1]],
                              sp[(cpw - 1) % 3]).wait()
        pltpu.sync_copy(den_v, den_hbm.at[cid].at[sid])
        plsc.subcore_barrier()

        for k in range((nblocks + _NS - 1) // _NS):
            b = sid + _NS * k
            @pl.when(b < nblocks)
            def _():
                pltpu.sync_copy(
                    p_sh.at[pl.ds(b * zrows, zrows)],
                    p_hbm.at[cid].at[pl.ds(b * zrows, zrows)])

    return pl.kernel(
        body,
        out_type=[
            jax.ShapeDtypeStruct((_NC, n_nodes, _MH), jnp.float32),
            jax.ShapeDtypeStruct((_NC, _NS, n_nodes), jnp.float32),
        ],
        mesh=mesh,
        scratch_types=[
            pltpu.VMEM((cpw, _CHUNK), jnp.int32),
            pltpu.VMEM((cpw, _CHUNK), jnp.int32),
            pltpu.VMEM((_CHUNK, d_sig), jnp.float32),
            pltpu.VMEM((_CHUNK, d_sig), jnp.float32),
            pltpu.VMEM((_CHUNK, _MH), jnp.float32),
            pltpu.VMEM((_CHUNK, _MH), jnp.float32),
            pltpu.VMEM((_CHUNK, _MH), jnp.float32),
            pltpu.VMEM((_CHUNK,), jnp.float32),
            pltpu.VMEM((n_nodes,), jnp.float32),
            pltpu.VMEM((200, _MH), jnp.float32),
            pltpu.VMEM_SHARED((n_nodes, _MH), jnp.float32),
        ] + [pltpu.SemaphoreType.DMA] * 8,
        compiler_params=pltpu.CompilerParams(
            needs_layout_passes=False, use_tc_tiling_on_sc=False),
    )


def _combine_body(h_ref, p0_ref, p1_ref, d0_ref, d1_ref, wh_ref, wc0_ref,
                  wc1_ref, ba_ref, wd_ref, bd_ref, out_ref):
    d0 = jnp.sum(d0_ref[...], axis=-1, keepdims=True)
    d1 = jnp.sum(d1_ref[...], axis=-1, keepdims=True)
    comm0 = p0_ref[...] / jnp.maximum(d0, 1e-30)
    comm1 = p1_ref[...] / jnp.maximum(d1, 1e-30)
    combined = jnp.maximum(
        jnp.dot(h_ref[...], wh_ref[...], preferred_element_type=jnp.float32)
        + jnp.dot(comm0, wc0_ref[...], preferred_element_type=jnp.float32)
        + jnp.dot(comm1, wc1_ref[...], preferred_element_type=jnp.float32)
        + ba_ref[...], 0.0)
    out_ref[...] = (
        jnp.dot(combined, wd_ref[...], preferred_element_type=jnp.float32)
        + bd_ref[...])


def _combine(h, p0, p1, d0, d1, W_agg, b_agg, W_dec, b_dec, block_rows):
    n, d_h = h.shape
    d_out = W_dec.shape[1]
    grid = (n // block_rows,)
    full = lambda shape: pl.BlockSpec(shape, lambda i: (0, 0))
    return pl.pallas_call(
        _combine_body,
        grid=grid,
        in_specs=[
            pl.BlockSpec((block_rows, d_h), lambda i: (i, 0)),
            pl.BlockSpec((block_rows, _MH), lambda i: (i, 0)),
            pl.BlockSpec((block_rows, _MH), lambda i: (i, 0)),
            pl.BlockSpec((block_rows, _NS), lambda i: (i, 0)),
            pl.BlockSpec((block_rows, _NS), lambda i: (i, 0)),
            full((d_h, d_h)), full((_MH, d_h)), full((_MH, d_h)),
            full((1, d_h)),
            full((d_h, d_out)), full((1, d_out)),
        ],
        out_specs=pl.BlockSpec((block_rows, d_out), lambda i: (i, 0)),
        out_shape=jax.ShapeDtypeStruct((n, d_out), jnp.float32),
    )(h, p0, p1, d0, d1, W_agg[:d_h], W_agg[d_h:d_h + _MH],
      W_agg[d_h + _MH:], b_agg.reshape(1, -1), W_dec, b_dec.reshape(1, -1))


@jax.jit
def kernel(x, edge_index, W_enc, b_enc, W_sig, b_sig, W_msg, b_msg,
           W_agg, b_agg, W_dec, b_dec):
    n = x.shape[0]
    e = edge_index.shape[1]
    d_sig = W_sig.shape[1]

    h, sig, msg0, msg1 = _encode(x, W_enc, b_enc, W_sig, b_sig, W_msg, b_msg,
                                 block_rows=400)

    slab = _CHUNK * _NS
    cpw = (e + slab - 1) // slab
    cpw += (-cpw) % 6
    e_pad = cpw * slab
    src = edge_index[0].astype(jnp.int32)
    dst = edge_index[1].astype(jnp.int32)
    pad = e_pad - e
    if pad:
        src = jnp.concatenate([src, jnp.zeros((pad,), jnp.int32)])
        dst = jnp.concatenate([dst, jnp.zeros((pad,), jnp.int32)])
    src3 = src.reshape(_NS, cpw, _CHUNK)
    dst3 = dst.reshape(_NS, cpw, _CHUNK)

    p, den = _sc_edge_kernel(n, e, cpw, d_sig)(sig, msg0, msg1, src3, dst3)
    den_t = jnp.swapaxes(den, 1, 2)

    return _combine(h, p[0], p[1], den_t[0], den_t[1], W_agg, b_agg,
                    W_dec, b_dec, block_rows=400)


# speedup vs baseline: 1.5485x; 1.5485x over previous
import jax
import jax.numpy as jnp
from jax import lax
from jax.experimental import pallas as pl
from jax.experimental.pallas import tpu as pltpu
from jax.experimental.pallas import tpu_sc as plsc

_NC = 2
_NS = 16
_L = 16
_CHUNK = 128
_MH = 64
_DW = _MH + _L


def _encode_body(x_ref, we_ref, be_ref, ws_ref, bs_ref, wm_ref, bm_ref,
                 h_ref, sig_ref, msg0_ref, msg1_ref):
    x = x_ref[...]
    h = jnp.dot(x, we_ref[...], preferred_element_type=jnp.float32) + be_ref[...]
    s = jnp.dot(h, ws_ref[...], preferred_element_type=jnp.float32) + bs_ref[...]
    nrm = jnp.sqrt(jnp.sum(s * s, axis=-1, keepdims=True))
    sig_ref[...] = s / jnp.maximum(nrm, 1e-12)
    msg = jnp.maximum(
        jnp.dot(h, wm_ref[...], preferred_element_type=jnp.float32) + bm_ref[...],
        0.0)
    msg0_ref[...] = msg[:, :_MH]
    msg1_ref[...] = msg[:, _MH:]
    h_ref[...] = h


def _encode(x, W_enc, b_enc, W_sig, b_sig, W_msg, b_msg, block_rows):
    n, d_in = x.shape
    d_h = W_enc.shape[1]
    d_sig = W_sig.shape[1]
    grid = (n // block_rows,)
    full = lambda shape: pl.BlockSpec(shape, lambda i: (0, 0))
    return pl.pallas_call(
        _encode_body,
        grid=grid,
        in_specs=[
            pl.BlockSpec((block_rows, d_in), lambda i: (i, 0)),
            full((d_in, d_h)), full((1, d_h)),
            full((d_h, d_sig)), full((1, d_sig)),
            full((d_h, d_h)), full((1, d_h)),
        ],
        out_specs=[
            pl.BlockSpec((block_rows, d_h), lambda i: (i, 0)),
            pl.BlockSpec((block_rows, d_sig), lambda i: (i, 0)),
            pl.BlockSpec((block_rows, _MH), lambda i: (i, 0)),
            pl.BlockSpec((block_rows, _MH), lambda i: (i, 0)),
        ],
        out_shape=[
            jax.ShapeDtypeStruct((n, d_h), jnp.float32),
            jax.ShapeDtypeStruct((n, d_sig), jnp.float32),
            jax.ShapeDtypeStruct((n, _MH), jnp.float32),
            jax.ShapeDtypeStruct((n, _MH), jnp.float32),
        ],
    )(x, W_enc, b_enc.reshape(1, -1), W_sig, b_sig.reshape(1, -1),
      W_msg, b_msg.reshape(1, -1))


def _sc_edge_kernel(n_nodes, n_edges, cpw, d_sig):
    zrows = 200
    nblocks = n_nodes // zrows
    mesh = plsc.VectorSubcoreMesh(core_axis_name="c", subcore_axis_name="s")

    def body(sig_hbm, msg0_hbm, msg1_hbm, src_hbm, dst_hbm, p_hbm, den_hbm,
             src_v, dst_v, ssrc0, sdst0, m0, m1, ex_v, den_v,
             zero_v,
             p_sh,
             ga0, gb0, gm0, gm1, sp0, sp1):
        cid = lax.axis_index("c")
        sid = lax.axis_index("s")
        lane = lax.broadcasted_iota(jnp.int32, (_L,), 0)
        msgb = (m0, m1)
        gm = (gm0, gm1)
        sp = (sp0, sp1)

        def zfill(r, _):
            z = jnp.zeros((_L,), jnp.float32)
            for q in range(_MH // _L):
                zero_v[r, pl.ds(q * _L, _L)] = z
            return 0
        lax.fori_loop(0, zrows, zfill, 0, unroll=False)

        def dfill(r, _):
            den_v[pl.ds(r * _L, _L)] = jnp.zeros((_L,), jnp.float32)
            return 0
        lax.fori_loop(0, n_nodes // _L, dfill, 0, unroll=False)

        for k in range((nblocks + _NS - 1) // _NS):
            b = sid + _NS * k
            @pl.when(b < nblocks)
            def _():
                pltpu.sync_copy(zero_v, p_sh.at[pl.ds(b * zrows, zrows)])
        plsc.subcore_barrier()

        pltpu.sync_copy(src_hbm.at[sid], src_v)
        pltpu.sync_copy(dst_hbm.at[sid], dst_v)

        ebase = sid * (cpw * _CHUNK)

        def issue_sig_gathers(j):
            pltpu.async_copy(sig_hbm.at[src_v.at[j]], ssrc0, ga0)
            pltpu.async_copy(sig_hbm.at[dst_v.at[j]], sdst0, gb0)

        def issue_msg_gather(j, m):
            @pl.when(cid == 0)
            def _():
                pltpu.async_copy(msg0_hbm.at[dst_v.at[j]], msgb[m], gm[m])
            @pl.when(cid == 1)
            def _():
                pltpu.async_copy(msg1_hbm.at[dst_v.at[j]], msgb[m], gm[m])

        issue_sig_gathers(0)
        issue_msg_gather(0, 0)

        def do_chunk(j, m):
            pltpu.make_async_copy(sig_hbm.at[src_v.at[j]], ssrc0,
                                  ga0).wait()
            pltpu.make_async_copy(sig_hbm.at[dst_v.at[j]], sdst0,
                                  gb0).wait()

            for g in range(_CHUNK // _L):
                rows = g * _L + lane
                acc = jnp.zeros((_L,), jnp.float32)
                for f in range(d_sig):
                    col = jnp.full((_L,), f, jnp.int32)
                    a = plsc.load_gather(ssrc0, [rows, col])
                    bb = plsc.load_gather(sdst0, [rows, col])
                    acc = acc + a * bb
                eid = ebase + j * _CHUNK + g * _L + lane
                ex = jnp.where(eid < n_edges, jnp.exp(acc), 0.0)
                ex_v[pl.ds(g * _L, _L)] = ex
                sidx = src_v[j, pl.ds(g * _L, _L)]
                plsc.addupdate_scatter(den_v, [sidx], ex)

            @pl.when(j + 1 < cpw)
            def _():
                issue_sig_gathers(j + 1)

            pm = 1 - m
            @pl.when(j >= 1)
            def _():
                pltpu.make_async_copy(msgb[pm], p_sh.at[src_v.at[j]],
                                      sp[pm]).wait()
            @pl.when(j + 1 < cpw)
            def _():
                issue_msg_gather(j + 1, pm)

            pltpu.make_async_copy(msg0_hbm.at[dst_v.at[j]], msgb[m],
                                  gm[m]).wait()

            def scale_row(r, _):
                spl = plsc.load_gather(ex_v, [jnp.full((_L,), r, jnp.int32)])
                for q in range(_MH // _L):
                    msgb[m][r, pl.ds(q * _L, _L)] = (
                        msgb[m][r, pl.ds(q * _L, _L)] * spl)
                return 0
            lax.fori_loop(0, _CHUNK, scale_row, 0, unroll=4)

            pltpu.async_copy(msgb[m], p_sh.at[src_v.at[j]], sp[m], add=True)

        def block_body(blk, _):
            do_chunk(2 * blk, 0)
            do_chunk(2 * blk + 1, 1)
            return 0

        lax.fori_loop(0, cpw // 2, block_body, 0, unroll=False)
        pltpu.make_async_copy(msgb[(cpw - 1) % 2],
                              p_sh.at[src_v.at[cpw - 1]],
                              sp[(cpw - 1) % 2]).wait()
        pltpu.sync_copy(den_v, den_hbm.at[cid].at[sid])
        plsc.subcore_barrier()

        for k in range((nblocks + _NS - 1) // _NS):
            b = sid + _NS * k
            @pl.when(b < nblocks)
            def _():
                pltpu.sync_copy(
                    p_sh.at[pl.ds(b * zrows, zrows)],
                    p_hbm.at[cid].at[pl.ds(b * zrows, zrows)])

    return pl.kernel(
        body,
        out_type=[
            jax.ShapeDtypeStruct((_NC, n_nodes, _MH), jnp.float32),
            jax.ShapeDtypeStruct((_NC, _NS, n_nodes), jnp.float32),
        ],
        mesh=mesh,
        scratch_types=[
            pltpu.VMEM((cpw, _CHUNK), jnp.int32),
            pltpu.VMEM((cpw, _CHUNK), jnp.int32),
            pltpu.VMEM((_CHUNK, d_sig), jnp.float32),
            pltpu.VMEM((_CHUNK, d_sig), jnp.float32),
            pltpu.VMEM((_CHUNK, _MH), jnp.float32),
            pltpu.VMEM((_CHUNK, _MH), jnp.float32),
            pltpu.VMEM((_CHUNK,), jnp.float32),
            pltpu.VMEM((n_nodes,), jnp.float32),
            pltpu.VMEM((200, _MH), jnp.float32),
            pltpu.VMEM_SHARED((n_nodes, _MH), jnp.float32),
        ] + [pltpu.SemaphoreType.DMA] * 6,
        compiler_params=pltpu.CompilerParams(
            needs_layout_passes=False, use_tc_tiling_on_sc=False),
    )


def _combine_body(h_ref, p0_ref, p1_ref, d0_ref, d1_ref, wh_ref, wc0_ref,
                  wc1_ref, ba_ref, wd_ref, bd_ref, out_ref):
    d0 = jnp.sum(d0_ref[...], axis=-1, keepdims=True)
    d1 = jnp.sum(d1_ref[...], axis=-1, keepdims=True)
    comm0 = p0_ref[...] / jnp.maximum(d0, 1e-30)
    comm1 = p1_ref[...] / jnp.maximum(d1, 1e-30)
    combined = jnp.maximum(
        jnp.dot(h_ref[...], wh_ref[...], preferred_element_type=jnp.float32)
        + jnp.dot(comm0, wc0_ref[...], preferred_element_type=jnp.float32)
        + jnp.dot(comm1, wc1_ref[...], preferred_element_type=jnp.float32)
        + ba_ref[...], 0.0)
    out_ref[...] = (
        jnp.dot(combined, wd_ref[...], preferred_element_type=jnp.float32)
        + bd_ref[...])


def _combine(h, p0, p1, d0, d1, W_agg, b_agg, W_dec, b_dec, block_rows):
    n, d_h = h.shape
    d_out = W_dec.shape[1]
    grid = (n // block_rows,)
    full = lambda shape: pl.BlockSpec(shape, lambda i: (0, 0))
    return pl.pallas_call(
        _combine_body,
        grid=grid,
        in_specs=[
            pl.BlockSpec((block_rows, d_h), lambda i: (i, 0)),
            pl.BlockSpec((block_rows, _MH), lambda i: (i, 0)),
            pl.BlockSpec((block_rows, _MH), lambda i: (i, 0)),
            pl.BlockSpec((block_rows, _NS), lambda i: (i, 0)),
            pl.BlockSpec((block_rows, _NS), lambda i: (i, 0)),
            full((d_h, d_h)), full((_MH, d_h)), full((_MH, d_h)),
            full((1, d_h)),
            full((d_h, d_out)), full((1, d_out)),
        ],
        out_specs=pl.BlockSpec((block_rows, d_out), lambda i: (i, 0)),
        out_shape=jax.ShapeDtypeStruct((n, d_out), jnp.float32),
    )(h, p0, p1, d0, d1, W_agg[:d_h], W_agg[d_h:d_h + _MH],
      W_agg[d_h + _MH:], b_agg.reshape(1, -1), W_dec, b_dec.reshape(1, -1))


@jax.jit
def kernel(x, edge_index, W_enc, b_enc, W_sig, b_sig, W_msg, b_msg,
           W_agg, b_agg, W_dec, b_dec):
    n = x.shape[0]
    e = edge_index.shape[1]
    d_sig = W_sig.shape[1]

    h, sig, msg0, msg1 = _encode(x, W_enc, b_enc, W_sig, b_sig, W_msg, b_msg,
                                 block_rows=400)

    slab = _CHUNK * _NS
    cpw = (e + slab - 1) // slab
    cpw += cpw % 2
    e_pad = cpw * slab
    src = edge_index[0].astype(jnp.int32)
    dst = edge_index[1].astype(jnp.int32)
    pad = e_pad - e
    if pad:
        src = jnp.concatenate([src, jnp.zeros((pad,), jnp.int32)])
        dst = jnp.concatenate([dst, jnp.zeros((pad,), jnp.int32)])
    src3 = src.reshape(_NS, cpw, _CHUNK)
    dst3 = dst.reshape(_NS, cpw, _CHUNK)

    p, den = _sc_edge_kernel(n, e, cpw, d_sig)(sig, msg0, msg1, src3, dst3)
    den_t = jnp.swapaxes(den, 1, 2)

    return _combine(h, p[0], p[1], den_t[0], den_t[1], W_agg, b_agg,
                    W_dec, b_dec, block_rows=400)


# speedup vs baseline: 1.5704x; 1.0141x over previous
import jax
import jax.numpy as jnp
from jax import lax
from jax.experimental import pallas as pl
from jax.experimental.pallas import tpu as pltpu
from jax.experimental.pallas import tpu_sc as plsc

_NC = 2
_NS = 16
_L = 16
_CHUNK = 128
_MH = 64
_DW = _MH + _L


def _encode_body(x_ref, we_ref, be_ref, ws_ref, bs_ref, wm_ref, bm_ref,
                 h_ref, sig_ref, msg0_ref, msg1_ref):
    x = x_ref[...]
    h = jnp.dot(x, we_ref[...], preferred_element_type=jnp.float32) + be_ref[...]
    s = jnp.dot(h, ws_ref[...], preferred_element_type=jnp.float32) + bs_ref[...]
    nrm = jnp.sqrt(jnp.sum(s * s, axis=-1, keepdims=True))
    sig_ref[...] = s / jnp.maximum(nrm, 1e-12)
    msg = jnp.maximum(
        jnp.dot(h, wm_ref[...], preferred_element_type=jnp.float32) + bm_ref[...],
        0.0)
    msg0_ref[...] = msg[:, :_MH]
    msg1_ref[...] = msg[:, _MH:]
    h_ref[...] = h


def _encode(x, W_enc, b_enc, W_sig, b_sig, W_msg, b_msg, block_rows):
    n, d_in = x.shape
    d_h = W_enc.shape[1]
    d_sig = W_sig.shape[1]
    grid = (n // block_rows,)
    full = lambda shape: pl.BlockSpec(shape, lambda i: (0, 0))
    return pl.pallas_call(
        _encode_body,
        grid=grid,
        in_specs=[
            pl.BlockSpec((block_rows, d_in), lambda i: (i, 0)),
            full((d_in, d_h)), full((1, d_h)),
            full((d_h, d_sig)), full((1, d_sig)),
            full((d_h, d_h)), full((1, d_h)),
        ],
        out_specs=[
            pl.BlockSpec((block_rows, d_h), lambda i: (i, 0)),
            pl.BlockSpec((block_rows, d_sig), lambda i: (i, 0)),
            pl.BlockSpec((block_rows, _MH), lambda i: (i, 0)),
            pl.BlockSpec((block_rows, _MH), lambda i: (i, 0)),
        ],
        out_shape=[
            jax.ShapeDtypeStruct((n, d_h), jnp.float32),
            jax.ShapeDtypeStruct((n, d_sig), jnp.float32),
            jax.ShapeDtypeStruct((n, _MH), jnp.float32),
            jax.ShapeDtypeStruct((n, _MH), jnp.float32),
        ],
    )(x, W_enc, b_enc.reshape(1, -1), W_sig, b_sig.reshape(1, -1),
      W_msg, b_msg.reshape(1, -1))


def _sc_edge_kernel(n_nodes, n_edges, cpw, d_sig):
    zrows = 200
    nblocks = n_nodes // zrows
    mesh = plsc.VectorSubcoreMesh(core_axis_name="c", subcore_axis_name="s")

    def body(sig_hbm, msg0_hbm, msg1_hbm, src_hbm, dst_hbm, p_hbm, den_hbm,
             src_v, dst_v, ssrc0, sdst0, m0, m1, ex_v, den_v,
             zero_v,
             p_sh,
             ga0, gb0, gm0, gm1, sp0, sp1):
        cid = lax.axis_index("c")
        sid = lax.axis_index("s")
        lane = lax.broadcasted_iota(jnp.int32, (_L,), 0)
        msgb = (m0, m1)
        gm = (gm0, gm1)
        sp = (sp0, sp1)

        pltpu.async_copy(src_hbm.at[sid], src_v, ga0)
        pltpu.async_copy(dst_hbm.at[sid], dst_v, gb0)

        def zfill(r, _):
            z = jnp.zeros((_L,), jnp.float32)
            for q in range(_MH // _L):
                zero_v[r, pl.ds(q * _L, _L)] = z
            return 0
        lax.fori_loop(0, zrows, zfill, 0, unroll=False)

        for k in range((nblocks + _NS - 1) // _NS):
            b = sid + _NS * k
            @pl.when(b < nblocks)
            def _():
                pltpu.async_copy(zero_v, p_sh.at[pl.ds(b * zrows, zrows)],
                                 sp0)

        def dfill(r, _):
            den_v[pl.ds(r * _L, _L)] = jnp.zeros((_L,), jnp.float32)
            return 0
        lax.fori_loop(0, n_nodes // _L, dfill, 0, unroll=False)

        pltpu.make_async_copy(src_hbm.at[sid], src_v, ga0).wait()
        pltpu.make_async_copy(dst_hbm.at[sid], dst_v, gb0).wait()
        for k in range((nblocks + _NS - 1) // _NS):
            b = sid + _NS * k
            @pl.when(b < nblocks)
            def _():
                pltpu.make_async_copy(
                    zero_v, p_sh.at[pl.ds(b * zrows, zrows)], sp0).wait()
        plsc.subcore_barrier()

        ebase = sid * (cpw * _CHUNK)

        def issue_sig_gathers(j):
            pltpu.async_copy(sig_hbm.at[src_v.at[j]], ssrc0, ga0)
            pltpu.async_copy(sig_hbm.at[dst_v.at[j]], sdst0, gb0)

        def issue_msg_gather(j, m):
            @pl.when(cid == 0)
            def _():
                pltpu.async_copy(msg0_hbm.at[dst_v.at[j]], msgb[m], gm[m])
            @pl.when(cid == 1)
            def _():
                pltpu.async_copy(msg1_hbm.at[dst_v.at[j]], msgb[m], gm[m])

        issue_sig_gathers(0)
        issue_msg_gather(0, 0)

        def do_chunk(j, m):
            pltpu.make_async_copy(sig_hbm.at[src_v.at[j]], ssrc0,
                                  ga0).wait()
            pltpu.make_async_copy(sig_hbm.at[dst_v.at[j]], sdst0,
                                  gb0).wait()

            for g in range(_CHUNK // _L):
                rows = g * _L + lane
                acc = jnp.zeros((_L,), jnp.float32)
                for f in range(d_sig):
                    col = jnp.full((_L,), f, jnp.int32)
                    a = plsc.load_gather(ssrc0, [rows, col])
                    bb = plsc.load_gather(sdst0, [rows, col])
                    acc = acc + a * bb
                eid = ebase + j * _CHUNK + g * _L + lane
                ex = jnp.where(eid < n_edges, jnp.exp(acc), 0.0)
                ex_v[pl.ds(g * _L, _L)] = ex
                sidx = src_v[j, pl.ds(g * _L, _L)]
                plsc.addupdate_scatter(den_v, [sidx], ex)

            @pl.when(j + 1 < cpw)
            def _():
                issue_sig_gathers(j + 1)

            pm = 1 - m
            @pl.when(j >= 1)
            def _():
                pltpu.make_async_copy(msgb[pm], p_sh.at[src_v.at[j]],
                                      sp[pm]).wait()
            @pl.when(j + 1 < cpw)
            def _():
                issue_msg_gather(j + 1, pm)

            pltpu.make_async_copy(msg0_hbm.at[dst_v.at[j]], msgb[m],
                                  gm[m]).wait()

            def scale_row(r, _):
                spl = plsc.load_gather(ex_v, [jnp.full((_L,), r, jnp.int32)])
                for q in range(_MH // _L):
                    msgb[m][r, pl.ds(q * _L, _L)] = (
                        msgb[m][r, pl.ds(q * _L, _L)] * spl)
                return 0
            lax.fori_loop(0, _CHUNK, scale_row, 0, unroll=4)

            pltpu.async_copy(msgb[m], p_sh.at[src_v.at[j]], sp[m], add=True)

        def block_body(blk, _):
            do_chunk(2 * blk, 0)
            do_chunk(2 * blk + 1, 1)
            return 0

        lax.fori_loop(0, cpw // 2, block_body, 0, unroll=False)
        pltpu.make_async_copy(msgb[(cpw - 1) % 2],
                              p_sh.at[src_v.at[cpw - 1]],
                              sp[(cpw - 1) % 2]).wait()
        pltpu.async_copy(den_v, den_hbm.at[cid].at[sid], ga0)
        plsc.subcore_barrier()

        for k in range((nblocks + _NS - 1) // _NS):
            b = sid + _NS * k
            @pl.when(b < nblocks)
            def _():
                pltpu.async_copy(
                    p_sh.at[pl.ds(b * zrows, zrows)],
                    p_hbm.at[cid].at[pl.ds(b * zrows, zrows)], sp0)
        for k in range((nblocks + _NS - 1) // _NS):
            b = sid + _NS * k
            @pl.when(b < nblocks)
            def _():
                pltpu.make_async_copy(
                    p_sh.at[pl.ds(b * zrows, zrows)],
                    p_hbm.at[cid].at[pl.ds(b * zrows, zrows)], sp0).wait()
        pltpu.make_async_copy(den_v, den_hbm.at[cid].at[sid], ga0).wait()

    return pl.kernel(
        body,
        out_type=[
            jax.ShapeDtypeStruct((_NC, n_nodes, _MH), jnp.float32),
            jax.ShapeDtypeStruct((_NC, _NS, n_nodes), jnp.float32),
        ],
        mesh=mesh,
        scratch_types=[
            pltpu.VMEM((cpw, _CHUNK), jnp.int32),
            pltpu.VMEM((cpw, _CHUNK), jnp.int32),
            pltpu.VMEM((_CHUNK, d_sig), jnp.float32),
            pltpu.VMEM((_CHUNK, d_sig), jnp.float32),
            pltpu.VMEM((_CHUNK, _MH), jnp.float32),
            pltpu.VMEM((_CHUNK, _MH), jnp.float32),
            pltpu.VMEM((_CHUNK,), jnp.float32),
            pltpu.VMEM((n_nodes,), jnp.float32),
            pltpu.VMEM((200, _MH), jnp.float32),
            pltpu.VMEM_SHARED((n_nodes, _MH), jnp.float32),
        ] + [pltpu.SemaphoreType.DMA] * 6,
        compiler_params=pltpu.CompilerParams(
            needs_layout_passes=False, use_tc_tiling_on_sc=False),
    )


def _combine_body(h_ref, p0_ref, p1_ref, d0_ref, d1_ref, wh_ref, wc0_ref,
                  wc1_ref, ba_ref, wd_ref, bd_ref, out_ref):
    d0 = jnp.sum(d0_ref[...], axis=-1, keepdims=True)
    d1 = jnp.sum(d1_ref[...], axis=-1, keepdims=True)
    comm0 = p0_ref[...] / jnp.maximum(d0, 1e-30)
    comm1 = p1_ref[...] / jnp.maximum(d1, 1e-30)
    combined = jnp.maximum(
        jnp.dot(h_ref[...], wh_ref[...], preferred_element_type=jnp.float32)
        + jnp.dot(comm0, wc0_ref[...], preferred_element_type=jnp.float32)
        + jnp.dot(comm1, wc1_ref[...], preferred_element_type=jnp.float32)
        + ba_ref[...], 0.0)
    out_ref[...] = (
        jnp.dot(combined, wd_ref[...], preferred_element_type=jnp.float32)
        + bd_ref[...])


def _combine(h, p0, p1, d0, d1, W_agg, b_agg, W_dec, b_dec, block_rows):
    n, d_h = h.shape
    d_out = W_dec.shape[1]
    grid = (n // block_rows,)
    full = lambda shape: pl.BlockSpec(shape, lambda i: (0, 0))
    return pl.pallas_call(
        _combine_body,
        grid=grid,
        in_specs=[
            pl.BlockSpec((block_rows, d_h), lambda i: (i, 0)),
            pl.BlockSpec((block_rows, _MH), lambda i: (i, 0)),
            pl.BlockSpec((block_rows, _MH), lambda i: (i, 0)),
            pl.BlockSpec((block_rows, _NS), lambda i: (i, 0)),
            pl.BlockSpec((block_rows, _NS), lambda i: (i, 0)),
            full((d_h, d_h)), full((_MH, d_h)), full((_MH, d_h)),
            full((1, d_h)),
            full((d_h, d_out)), full((1, d_out)),
        ],
        out_specs=pl.BlockSpec((block_rows, d_out), lambda i: (i, 0)),
        out_shape=jax.ShapeDtypeStruct((n, d_out), jnp.float32),
    )(h, p0, p1, d0, d1, W_agg[:d_h], W_agg[d_h:d_h + _MH],
      W_agg[d_h + _MH:], b_agg.reshape(1, -1), W_dec, b_dec.reshape(1, -1))


@jax.jit
def kernel(x, edge_index, W_enc, b_enc, W_sig, b_sig, W_msg, b_msg,
           W_agg, b_agg, W_dec, b_dec):
    n = x.shape[0]
    e = edge_index.shape[1]
    d_sig = W_sig.shape[1]

    h, sig, msg0, msg1 = _encode(x, W_enc, b_enc, W_sig, b_sig, W_msg, b_msg,
                                 block_rows=400)

    slab = _CHUNK * _NS
    cpw = (e + slab - 1) // slab
    cpw += cpw % 2
    e_pad = cpw * slab
    src = edge_index[0].astype(jnp.int32)
    dst = edge_index[1].astype(jnp.int32)
    pad = e_pad - e
    if pad:
        src = jnp.concatenate([src, jnp.zeros((pad,), jnp.int32)])
        dst = jnp.concatenate([dst, jnp.zeros((pad,), jnp.int32)])
    src3 = src.reshape(_NS, cpw, _CHUNK)
    dst3 = dst.reshape(_NS, cpw, _CHUNK)

    p, den = _sc_edge_kernel(n, e, cpw, d_sig)(sig, msg0, msg1, src3, dst3)
    den_t = jnp.swapaxes(den, 1, 2)

    return _combine(h, p[0], p[1], den_t[0], den_t[1], W_agg, b_agg,
                    W_dec, b_dec, block_rows=400)
